# baseline (speedup 1.0000x reference)
import functools

import jax
import jax.numpy as jnp
from jax import lax
from jax.experimental import pallas as pl
from jax.experimental.pallas import tpu as pltpu
from jax.experimental.pallas import tpu_sc as plsc

N = 50000
E = 800000
D_IN = 128
H = 64
HH = H // 2
EPS = 1e-5

BN_BLK = 5000
N_BLKS = N // BN_BLK

NC = 2
NS = 16
CH = 128
NCH_TOT = E // CH
CPT = 391
CPT_LAST = NCH_TOT - (NS - 1) * CPT
G = 16
NG = (CPT + G - 1) // G
PADC = 16
NBUF = 5
DPRE = 3
WPT = 3128
N_PAD = NS * WPT
ACC_ROWS = N_PAD
TRASH = N


def _pre_body(x_ref, w_ref, rw_ref, rb_ref, out_ref):
    x = x_ref[...]
    h = jnp.dot(x, w_ref[...], preferred_element_type=jnp.float32)
    r = jnp.dot(x, rw_ref[...], preferred_element_type=jnp.float32) + rb_ref[...]
    out_ref[...] = jnp.concatenate([h, jnp.maximum(r, 0.0)], axis=1)


def _pre_stage(x, W1, resW1, resb1):
    return pl.pallas_call(
        _pre_body,
        grid=(N_BLKS,),
        in_specs=[
            pl.BlockSpec((BN_BLK, D_IN), lambda i: (i, 0)),
            pl.BlockSpec((D_IN, H), lambda i: (0, 0)),
            pl.BlockSpec((D_IN, H), lambda i: (0, 0)),
            pl.BlockSpec((1, H), lambda i: (0, 0)),
        ],
        out_specs=pl.BlockSpec((BN_BLK, 128), lambda i: (i, 0)),
        out_shape=jax.ShapeDtypeStruct((N, 128), jnp.float32),
    )(x, W1, resW1, resb1.reshape(1, H))


def _finish_stats(t, i, g_ref, bt_ref, sc_ref, sh_ref, ps, pss):
    @pl.when(i == 0)
    def _():
        ps[...] = jnp.zeros_like(ps)
        pss[...] = jnp.zeros_like(pss)

    ps[...] += jnp.broadcast_to(jnp.sum(t, axis=0, keepdims=True), ps.shape)
    pss[...] += jnp.broadcast_to(jnp.sum(t * t, axis=0, keepdims=True), pss.shape)

    @pl.when(i == N_BLKS - 1)
    def _():
        mean = ps[...] * (1.0 / N)
        var = pss[...] * (1.0 / N) - mean * mean
        scale = g_ref[...] * lax.rsqrt(var + EPS)
        sc_ref[...] = scale
        sh_ref[...] = bt_ref[...] - mean * scale


def _post1_body(agg_ref, b_ref, res_ref, g_ref, bt_ref,
                t_ref, sc_ref, sh_ref, ps, pss):
    i = pl.program_id(0)
    agg = agg_ref[:, :H]
    t = jnp.maximum(agg + b_ref[...], 0.0) + res_ref[:, H:]
    t_ref[...] = t
    _finish_stats(t, i, g_ref, bt_ref, sc_ref, sh_ref, ps, pss)


def _post1_stage(aggcat, b1, res1, gamma, beta):
    return pl.pallas_call(
        _post1_body,
        grid=(N_BLKS,),
        in_specs=[
            pl.BlockSpec((BN_BLK, 128), lambda i: (i, 0)),
            pl.BlockSpec((1, H), lambda i: (0, 0)),
            pl.BlockSpec((BN_BLK, 128), lambda i: (i, 0)),
            pl.BlockSpec((8, H), lambda i: (0, 0)),
            pl.BlockSpec((8, H), lambda i: (0, 0)),
        ],
        out_specs=[
            pl.BlockSpec((BN_BLK, H), lambda i: (i, 0)),
            pl.BlockSpec((8, H), lambda i: (0, 0)),
            pl.BlockSpec((8, H), lambda i: (0, 0)),
        ],
        out_shape=[
            jax.ShapeDtypeStruct((N, H), jnp.float32),
            jax.ShapeDtypeStruct((8, H), jnp.float32),
            jax.ShapeDtypeStruct((8, H), jnp.float32),
        ],
        scratch_shapes=[
            pltpu.VMEM((8, H), jnp.float32),
            pltpu.VMEM((8, H), jnp.float32),
        ],
    )(aggcat, b1.reshape(1, H),
      res1, _bcast8(gamma), _bcast8(beta))


def _post2_body(agg_ref, w_ref, b_ref, res_ref, g_ref, bt_ref,
                t_ref, sc_ref, sh_ref, ps, pss):
    i = pl.program_id(0)
    agg = agg_ref[:, :H]
    conv = jnp.dot(agg, w_ref[...], preferred_element_type=jnp.float32) + b_ref[...]
    t = jnp.maximum(conv, 0.0) + res_ref[:, H:]
    t_ref[...] = t
    _finish_stats(t, i, g_ref, bt_ref, sc_ref, sh_ref, ps, pss)


def _post2_stage(aggcat, W2, b2, res2, gamma, beta):
    return pl.pallas_call(
        _post2_body,
        grid=(N_BLKS,),
        in_specs=[
            pl.BlockSpec((BN_BLK, 128), lambda i: (i, 0)),
            pl.BlockSpec((H, H), lambda i: (0, 0)),
            pl.BlockSpec((1, H), lambda i: (0, 0)),
            pl.BlockSpec((BN_BLK, 128), lambda i: (i, 0)),
            pl.BlockSpec((8, H), lambda i: (0, 0)),
            pl.BlockSpec((8, H), lambda i: (0, 0)),
        ],
        out_specs=[
            pl.BlockSpec((BN_BLK, H), lambda i: (i, 0)),
            pl.BlockSpec((8, H), lambda i: (0, 0)),
            pl.BlockSpec((8, H), lambda i: (0, 0)),
        ],
        out_shape=[
            jax.ShapeDtypeStruct((N, H), jnp.float32),
            jax.ShapeDtypeStruct((8, H), jnp.float32),
            jax.ShapeDtypeStruct((8, H), jnp.float32),
        ],
        scratch_shapes=[
            pltpu.VMEM((8, H), jnp.float32),
            pltpu.VMEM((8, H), jnp.float32),
        ],
    )(aggcat, W2, b2.reshape(1, H), res2, _bcast8(gamma), _bcast8(beta))


def _bcast8(v):
    return jnp.broadcast_to(v.reshape(1, H), (8, H))


def _norm_split_body(t_ref, sc_ref, sh_ref, rw_ref, rb_ref, out_ref):
    h = t_ref[...] * sc_ref[0:1, :] + sh_ref[0:1, :]
    r = jnp.dot(h, rw_ref[...], preferred_element_type=jnp.float32) + rb_ref[...]
    out_ref[...] = jnp.concatenate([h, jnp.maximum(r, 0.0)], axis=1)


def _norm_split_stage(t1, scale, shift, resW2, resb2):
    return pl.pallas_call(
        _norm_split_body,
        grid=(N_BLKS,),
        in_specs=[
            pl.BlockSpec((BN_BLK, H), lambda i: (i, 0)),
            pl.BlockSpec((8, H), lambda i: (0, 0)),
            pl.BlockSpec((8, H), lambda i: (0, 0)),
            pl.BlockSpec((H, H), lambda i: (0, 0)),
            pl.BlockSpec((1, H), lambda i: (0, 0)),
        ],
        out_specs=pl.BlockSpec((BN_BLK, 128), lambda i: (i, 0)),
        out_shape=jax.ShapeDtypeStruct((N, 128), jnp.float32),
    )(t1, scale, shift, resW2, resb2.reshape(1, H))


def _norm_body(t_ref, sc_ref, sh_ref, out_ref):
    out_ref[...] = t_ref[...] * sc_ref[0:1, :] + sh_ref[0:1, :]


def _norm_stage(t2, scale, shift):
    return pl.pallas_call(
        _norm_body,
        grid=(N_BLKS,),
        in_specs=[
            pl.BlockSpec((BN_BLK, H), lambda i: (i, 0)),
            pl.BlockSpec((8, H), lambda i: (0, 0)),
            pl.BlockSpec((8, H), lambda i: (0, 0)),
        ],
        out_specs=pl.BlockSpec((BN_BLK, H), lambda i: (i, 0)),
        out_shape=jax.ShapeDtypeStruct((N, H), jnp.float32),
    )(t2, scale, shift)


RPT = N // NS


def _seg_body(hcat, eidx, zeros, out, hs, acc, idx, rows, gsem, ssem, isem):
    c = lax.axis_index("c")
    s = lax.axis_index("s")
    base = s * CPT
    cnt = jnp.where(s == NS - 1, CPT_LAST, CPT)
    pltpu.sync_copy(hcat.at[pl.ds(s * RPT, RPT), pl.ds(c * HH, HH)],
                    hs.at[c, pl.ds(s * RPT, RPT)])
    hc = hs.at[c]
    pltpu.sync_copy(zeros, acc.at[pl.ds(s * WPT, WPT)])
    pltpu.sync_copy(eidx.at[0, pl.ds(base, G)], idx.at[0, 0])
    pltpu.sync_copy(eidx.at[1, pl.ds(base, G)], idx.at[0, 1])
    plsc.subcore_barrier()

    def _wait_rows(sem, b):
        pltpu.make_async_copy(hc.at[pl.ds(0, CH)], rows.at[b], sem).wait()

    def _stage(g, slot):
        start = base + g * G
        pltpu.async_copy(eidx.at[0, pl.ds(start, G)], idx.at[slot, 0], isem)
        pltpu.async_copy(eidx.at[1, pl.ds(start, G)], idx.at[slot, 1], isem)

    for b in range(DPRE):
        pltpu.async_copy(hc.at[idx.at[0, 0, b]], rows.at[b], gsem)

    def body(t, carry):
        g = t // G
        r = t - g * G
        slot = lax.rem(g, 2)
        b = lax.rem(t, NBUF)

        @pl.when(jnp.logical_and(r == 0, g + 1 < NG))
        def _():
            _stage(g + 1, lax.rem(g + 1, 2))

        _wait_rows(gsem, b)
        pltpu.async_copy(rows.at[b], acc.at[idx.at[slot, 1, r]], ssem, add=True)

        @pl.when(t + DPRE < cnt)
        def _():
            @pl.when(t >= NBUF - DPRE)
            def _():
                _wait_rows(ssem, 0)

            td = t + DPRE
            gd = td // G
            rd = td - gd * G

            @pl.when(rd == 0)
            def _():
                pltpu.make_async_copy(eidx.at[pl.ds(0, 2), pl.ds(0, G)],
                                      idx.at[0], isem).wait()

            pltpu.async_copy(hc.at[idx.at[lax.rem(gd, 2), 0, rd]],
                             rows.at[lax.rem(td, NBUF)], gsem)

        return carry

    lax.fori_loop(0, cnt, body, 0)
    for _ in range(NBUF):
        _wait_rows(ssem, 0)
    plsc.subcore_barrier()
    pltpu.sync_copy(acc.at[pl.ds(s * WPT, WPT)],
                    out.at[pl.ds(s * WPT, WPT), pl.ds(c * HH, HH)])


@functools.lru_cache(maxsize=1)
def _make_seg_kernel():
    return pl.kernel(
        _seg_body,
        out_type=(jax.ShapeDtypeStruct((N_PAD, 128), jnp.float32),
                  jax.ShapeDtypeStruct((2, N, HH), jnp.float32)),
        mesh=plsc.VectorSubcoreMesh(
            core_axis_name="c", subcore_axis_name="s",
            num_cores=NC, num_subcores=NS,
        ),
        scratch_types=[
            pltpu.VMEM_SHARED((ACC_ROWS, HH), jnp.float32),
            pltpu.VMEM((2, 2, G, CH), jnp.int32),
            pltpu.VMEM((NBUF, CH, HH), jnp.float32),
            pltpu.SemaphoreType.DMA,
            pltpu.SemaphoreType.DMA,
            pltpu.SemaphoreType.DMA,
        ],
        compiler_params=pltpu.CompilerParams(use_tc_tiling_on_sc=False),
    )


def _segment_sum(hcat, eidx, zeros):
    agg, _ = _make_seg_kernel()(hcat, eidx, zeros)
    return agg


def kernel(x, edge_index, W1, b1, resW1, resb1, gamma1, beta1,
           W2, b2, resW2, resb2, gamma2, beta2):
    eidx = jnp.pad(edge_index, ((0, 0), (0, PADC * CH)))
    eidx = eidx.reshape(2, NCH_TOT + PADC, CH)
    zeros = jnp.zeros((WPT, HH), jnp.float32)

    hcat = _pre_stage(x, W1, resW1, resb1)
    agg1 = _segment_sum(hcat, eidx, zeros)
    t1, scale1, shift1 = _post1_stage(agg1, b1, hcat, gamma1, beta1)

    h1cat = _norm_split_stage(t1, scale1, shift1, resW2, resb2)
    agg2 = _segment_sum(h1cat, eidx, zeros)
    t2, scale2, shift2 = _post2_stage(agg2, W2, b2, h1cat, gamma2, beta2)
    return _norm_stage(t2, scale2, shift2)


# speedup vs baseline: 2.1031x; 2.1031x over previous
import functools

import jax
import jax.numpy as jnp
from jax import lax
from jax.experimental import pallas as pl
from jax.experimental.pallas import tpu as pltpu
from jax.experimental.pallas import tpu_sc as plsc

N = 50000
E = 800000
D_IN = 128
H = 64
HH = H // 2
EPS = 1e-5

BN_BLK = 5000
N_BLKS = N // BN_BLK

NC = 2
NS = 16
CH = 128
NCH_TOT = E // CH
CPT = 391
CPT_LAST = NCH_TOT - (NS - 1) * CPT
G = 16
NG = (CPT + G - 1) // G
PADC = 16
NBUF = 5
DPRE = 3
WPT = 3128
N_PAD = NS * WPT
ACC_ROWS = N_PAD
TRASH = N


def _pre_body(x_ref, w_ref, hcat_ref):
    h = jnp.dot(x_ref[...], w_ref[...], preferred_element_type=jnp.float32)
    hcat_ref[0] = h[:, :HH]
    hcat_ref[1] = h[:, HH:]


def _pre_stage(x, W1):
    return pl.pallas_call(
        _pre_body,
        grid=(N_BLKS,),
        in_specs=[
            pl.BlockSpec((BN_BLK, D_IN), lambda i: (i, 0)),
            pl.BlockSpec((D_IN, H), lambda i: (0, 0)),
        ],
        out_specs=pl.BlockSpec((2, BN_BLK, HH), lambda i: (0, i, 0)),
        out_shape=jax.ShapeDtypeStruct((2, N, HH), jnp.float32),
    )(x, W1)


def _res1_body(x_ref, rw_ref, rb_ref, res_ref):
    r = jnp.dot(x_ref[...], rw_ref[...],
                preferred_element_type=jnp.float32) + rb_ref[...]
    res_ref[...] = jnp.maximum(r, 0.0)


def _res1_stage(x, resW1, resb1):
    return pl.pallas_call(
        _res1_body,
        grid=(N_BLKS,),
        in_specs=[
            pl.BlockSpec((BN_BLK, D_IN), lambda i: (i, 0)),
            pl.BlockSpec((D_IN, H), lambda i: (0, 0)),
            pl.BlockSpec((1, H), lambda i: (0, 0)),
        ],
        out_specs=pl.BlockSpec((BN_BLK, H), lambda i: (i, 0)),
        out_shape=jax.ShapeDtypeStruct((N, H), jnp.float32),
    )(x, resW1, resb1.reshape(1, H))


def _finish_stats(t, i, g_ref, bt_ref, sc_ref, sh_ref, ps, pss):
    @pl.when(i == 0)
    def _():
        ps[...] = jnp.zeros_like(ps)
        pss[...] = jnp.zeros_like(pss)

    ps[...] += jnp.broadcast_to(jnp.sum(t, axis=0, keepdims=True), ps.shape)
    pss[...] += jnp.broadcast_to(jnp.sum(t * t, axis=0, keepdims=True), pss.shape)

    @pl.when(i == N_BLKS - 1)
    def _():
        mean = ps[...] * (1.0 / N)
        var = pss[...] * (1.0 / N) - mean * mean
        scale = g_ref[...] * lax.rsqrt(var + EPS)
        sc_ref[...] = scale
        sh_ref[...] = bt_ref[...] - mean * scale


def _post1_body(agg_ref, b_ref, res_ref, g_ref, bt_ref,
                t_ref, sc_ref, sh_ref, ps, pss):
    i = pl.program_id(0)
    agg = agg_ref[:, :H]
    t = jnp.maximum(agg + b_ref[...], 0.0) + res_ref[...]
    t_ref[...] = t
    _finish_stats(t, i, g_ref, bt_ref, sc_ref, sh_ref, ps, pss)


def _post1_stage(aggcat, b1, res1, gamma, beta):
    return pl.pallas_call(
        _post1_body,
        grid=(N_BLKS,),
        in_specs=[
            pl.BlockSpec((BN_BLK, 128), lambda i: (i, 0)),
            pl.BlockSpec((1, H), lambda i: (0, 0)),
            pl.BlockSpec((BN_BLK, H), lambda i: (i, 0)),
            pl.BlockSpec((8, H), lambda i: (0, 0)),
            pl.BlockSpec((8, H), lambda i: (0, 0)),
        ],
        out_specs=[
            pl.BlockSpec((BN_BLK, H), lambda i: (i, 0)),
            pl.BlockSpec((8, H), lambda i: (0, 0)),
            pl.BlockSpec((8, H), lambda i: (0, 0)),
        ],
        out_shape=[
            jax.ShapeDtypeStruct((N, H), jnp.float32),
            jax.ShapeDtypeStruct((8, H), jnp.float32),
            jax.ShapeDtypeStruct((8, H), jnp.float32),
        ],
        scratch_shapes=[
            pltpu.VMEM((8, H), jnp.float32),
            pltpu.VMEM((8, H), jnp.float32),
        ],
    )(aggcat, b1.reshape(1, H),
      res1, _bcast8(gamma), _bcast8(beta))


def _post2_body(agg_ref, w_ref, b_ref, res_ref, g_ref, bt_ref,
                t_ref, sc_ref, sh_ref, ps, pss):
    i = pl.program_id(0)
    agg = agg_ref[:, :H]
    conv = jnp.dot(agg, w_ref[...], preferred_element_type=jnp.float32) + b_ref[...]
    t = jnp.maximum(conv, 0.0) + res_ref[...]
    t_ref[...] = t
    _finish_stats(t, i, g_ref, bt_ref, sc_ref, sh_ref, ps, pss)


def _post2_stage(aggcat, W2, b2, res2, gamma, beta):
    return pl.pallas_call(
        _post2_body,
        grid=(N_BLKS,),
        in_specs=[
            pl.BlockSpec((BN_BLK, 128), lambda i: (i, 0)),
            pl.BlockSpec((H, H), lambda i: (0, 0)),
            pl.BlockSpec((1, H), lambda i: (0, 0)),
            pl.BlockSpec((BN_BLK, H), lambda i: (i, 0)),
            pl.BlockSpec((8, H), lambda i: (0, 0)),
            pl.BlockSpec((8, H), lambda i: (0, 0)),
        ],
        out_specs=[
            pl.BlockSpec((BN_BLK, H), lambda i: (i, 0)),
            pl.BlockSpec((8, H), lambda i: (0, 0)),
            pl.BlockSpec((8, H), lambda i: (0, 0)),
        ],
        out_shape=[
            jax.ShapeDtypeStruct((N, H), jnp.float32),
            jax.ShapeDtypeStruct((8, H), jnp.float32),
            jax.ShapeDtypeStruct((8, H), jnp.float32),
        ],
        scratch_shapes=[
            pltpu.VMEM((8, H), jnp.float32),
            pltpu.VMEM((8, H), jnp.float32),
        ],
    )(aggcat, W2, b2.reshape(1, H), res2, _bcast8(gamma), _bcast8(beta))


def _bcast8(v):
    return jnp.broadcast_to(v.reshape(1, H), (8, H))


def _norm_split_body(t_ref, sc_ref, sh_ref, hcat_ref):
    h = t_ref[...] * sc_ref[0:1, :] + sh_ref[0:1, :]
    hcat_ref[0] = h[:, :HH]
    hcat_ref[1] = h[:, HH:]


def _norm_split_stage(t1, scale, shift):
    return pl.pallas_call(
        _norm_split_body,
        grid=(N_BLKS,),
        in_specs=[
            pl.BlockSpec((BN_BLK, H), lambda i: (i, 0)),
            pl.BlockSpec((8, H), lambda i: (0, 0)),
            pl.BlockSpec((8, H), lambda i: (0, 0)),
        ],
        out_specs=pl.BlockSpec((2, BN_BLK, HH), lambda i: (0, i, 0)),
        out_shape=jax.ShapeDtypeStruct((2, N, HH), jnp.float32),
    )(t1, scale, shift)


def _res2_body(t_ref, sc_ref, sh_ref, rw_ref, rb_ref, res_ref):
    h = t_ref[...] * sc_ref[0:1, :] + sh_ref[0:1, :]
    r = jnp.dot(h, rw_ref[...], preferred_element_type=jnp.float32) + rb_ref[...]
    res_ref[...] = jnp.maximum(r, 0.0)


def _res2_stage(t1, scale, shift, resW2, resb2):
    return pl.pallas_call(
        _res2_body,
        grid=(N_BLKS,),
        in_specs=[
            pl.BlockSpec((BN_BLK, H), lambda i: (i, 0)),
            pl.BlockSpec((8, H), lambda i: (0, 0)),
            pl.BlockSpec((8, H), lambda i: (0, 0)),
            pl.BlockSpec((H, H), lambda i: (0, 0)),
            pl.BlockSpec((1, H), lambda i: (0, 0)),
        ],
        out_specs=pl.BlockSpec((BN_BLK, H), lambda i: (i, 0)),
        out_shape=jax.ShapeDtypeStruct((N, H), jnp.float32),
    )(t1, scale, shift, resW2, resb2.reshape(1, H))


def _norm_body(t_ref, sc_ref, sh_ref, out_ref):
    out_ref[...] = t_ref[...] * sc_ref[0:1, :] + sh_ref[0:1, :]


def _norm_stage(t2, scale, shift):
    return pl.pallas_call(
        _norm_body,
        grid=(N_BLKS,),
        in_specs=[
            pl.BlockSpec((BN_BLK, H), lambda i: (i, 0)),
            pl.BlockSpec((8, H), lambda i: (0, 0)),
            pl.BlockSpec((8, H), lambda i: (0, 0)),
        ],
        out_specs=pl.BlockSpec((BN_BLK, H), lambda i: (i, 0)),
        out_shape=jax.ShapeDtypeStruct((N, H), jnp.float32),
    )(t2, scale, shift)


def _seg_body(hcat, eidx, zeros, out, acc, idx, rows, gsem, ssem, isem):
    c = lax.axis_index("c")
    s = lax.axis_index("s")
    base = s * CPT
    cnt = jnp.where(s == NS - 1, CPT_LAST, CPT)
    hc = hcat.at[c]
    pltpu.sync_copy(zeros, acc.at[pl.ds(s * WPT, WPT)])
    pltpu.sync_copy(eidx.at[0, pl.ds(base, G)], idx.at[0, 0])
    pltpu.sync_copy(eidx.at[1, pl.ds(base, G)], idx.at[0, 1])
    plsc.subcore_barrier()

    def _wait_rows(sem, b):
        pltpu.make_async_copy(hc.at[pl.ds(0, CH)], rows.at[b], sem).wait()

    def _stage(g, slot):
        start = base + g * G
        pltpu.async_copy(eidx.at[0, pl.ds(start, G)], idx.at[slot, 0], isem)
        pltpu.async_copy(eidx.at[1, pl.ds(start, G)], idx.at[slot, 1], isem)

    for b in range(DPRE):
        pltpu.async_copy(hc.at[idx.at[0, 0, b]], rows.at[b], gsem)

    def body(t, carry):
        g = t // G
        r = t - g * G
        slot = lax.rem(g, 2)
        b = lax.rem(t, NBUF)

        @pl.when(jnp.logical_and(r == 0, g + 1 < NG))
        def _():
            _stage(g + 1, lax.rem(g + 1, 2))

        _wait_rows(gsem, b)
        pltpu.async_copy(rows.at[b], acc.at[idx.at[slot, 1, r]], ssem, add=True)

        @pl.when(t + DPRE < cnt)
        def _():
            @pl.when(t >= NBUF - DPRE)
            def _():
                _wait_rows(ssem, 0)

            td = t + DPRE
            gd = td // G
            rd = td - gd * G

            @pl.when(rd == 0)
            def _():
                pltpu.make_async_copy(eidx.at[pl.ds(0, 2), pl.ds(0, G)],
                                      idx.at[0], isem).wait()

            pltpu.async_copy(hc.at[idx.at[lax.rem(gd, 2), 0, rd]],
                             rows.at[lax.rem(td, NBUF)], gsem)

        return carry

    lax.fori_loop(0, cnt, body, 0)
    for _ in range(NBUF):
        _wait_rows(ssem, 0)
    plsc.subcore_barrier()
    pltpu.sync_copy(acc.at[pl.ds(s * WPT, WPT)],
                    out.at[pl.ds(s * WPT, WPT), pl.ds(c * HH, HH)])


@functools.lru_cache(maxsize=1)
def _make_seg_kernel():
    return pl.kernel(
        _seg_body,
        out_type=jax.ShapeDtypeStruct((N_PAD, 128), jnp.float32),
        mesh=plsc.VectorSubcoreMesh(
            core_axis_name="c", subcore_axis_name="s",
            num_cores=NC, num_subcores=NS,
        ),
        scratch_types=[
            pltpu.VMEM_SHARED((ACC_ROWS, HH), jnp.float32),
            pltpu.VMEM((2, 2, G, CH), jnp.int32),
            pltpu.VMEM((NBUF, CH, HH), jnp.float32),
            pltpu.SemaphoreType.DMA,
            pltpu.SemaphoreType.DMA,
            pltpu.SemaphoreType.DMA,
        ],
        compiler_params=pltpu.CompilerParams(use_tc_tiling_on_sc=False),
    )


def _segment_sum(hcat, eidx, zeros):
    return _make_seg_kernel()(hcat, eidx, zeros)


def kernel(x, edge_index, W1, b1, resW1, resb1, gamma1, beta1,
           W2, b2, resW2, resb2, gamma2, beta2):
    eidx = jnp.pad(edge_index, ((0, 0), (0, PADC * CH)))
    eidx = eidx.reshape(2, NCH_TOT + PADC, CH)
    zeros = jnp.zeros((WPT, HH), jnp.float32)

    hcat = _pre_stage(x, W1)
    agg1 = _segment_sum(hcat, eidx, zeros)
    res1 = _res1_stage(x, resW1, resb1)
    t1, scale1, shift1 = _post1_stage(agg1, b1, res1, gamma1, beta1)

    h1cat = _norm_split_stage(t1, scale1, shift1)
    agg2 = _segment_sum(h1cat, eidx, zeros)
    res2 = _res2_stage(t1, scale1, shift1, resW2, resb2)
    t2, scale2, shift2 = _post2_stage(agg2, W2, b2, res2, gamma2, beta2)
    return _norm_stage(t2, scale2, shift2)
